# Initial kernel scaffold; baseline (speedup 1.0000x reference)
import jax
import jax.numpy as jnp
from jax.experimental import pallas as pl


def kernel(embs, prev_h, batch_idx, W, b):
    raise NotImplementedError("write your pallas kernel here")


# speedup vs baseline: 4.9413x; 4.9413x over previous
import functools

import jax
import jax.numpy as jnp
from jax import lax
from jax.experimental import pallas as pl
from jax.experimental.pallas import tpu as pltpu
from jax.experimental.pallas import tpu_sc as plsc

NSEG = 1024
N = 320000
D = 128
NC = 2
NS = 16
NW = NC * NS
ROWS_PER_W = N // NW
CHUNK = 80
NCHUNK = ROWS_PER_W // CHUNK
SEG_PER_TILE = NSEG // NS


def _sc_body(prev_hbm, embs_hbm, idx_hbm, out_hbm,
             pbuf, ebuf, ibuf, zbuf, acc_p, acc_e,
             sem_p, sem_e, sem_i):
    core = lax.axis_index("c")
    sid = lax.axis_index("s")
    wid = core * NS + sid
    row_base = wid * ROWS_PER_W

    zeros16 = jnp.zeros((16,), jnp.float32)

    def zero_row(r, _):
        for c in range(D // 16):
            zbuf[r, pl.ds(c * 16, 16)] = zeros16
        return _

    lax.fori_loop(0, SEG_PER_TILE, zero_row, None)
    seg_lo = sid * SEG_PER_TILE
    pltpu.sync_copy(zbuf, acc_p.at[pl.ds(seg_lo, SEG_PER_TILE)])
    pltpu.sync_copy(zbuf, acc_e.at[pl.ds(seg_lo, SEG_PER_TILE)])
    plsc.subcore_barrier()

    def chunk_body(i, _):
        base = row_base + i * CHUNK
        cp = pltpu.async_copy(prev_hbm.at[pl.ds(base, CHUNK)], pbuf.at[0], sem_p)
        ce = pltpu.async_copy(embs_hbm.at[pl.ds(base, CHUNK)], ebuf.at[0], sem_e)
        ci = pltpu.async_copy(idx_hbm.at[pl.ds(base, CHUNK)], ibuf.at[0], sem_i)
        cp.wait()
        ce.wait()
        ci.wait()
        pltpu.sync_copy(pbuf.at[0], acc_p.at[ibuf.at[0]], add=True)
        pltpu.sync_copy(ebuf.at[0], acc_e.at[ibuf.at[0]], add=True)
        return _

    lax.fori_loop(0, NCHUNK, chunk_body, None)

    plsc.subcore_barrier()
    pltpu.sync_copy(acc_p.at[pl.ds(seg_lo, SEG_PER_TILE)],
                    out_hbm.at[core, 0, pl.ds(seg_lo, SEG_PER_TILE)])
    pltpu.sync_copy(acc_e.at[pl.ds(seg_lo, SEG_PER_TILE)],
                    out_hbm.at[core, 1, pl.ds(seg_lo, SEG_PER_TILE)])


_sc_segment_sums = functools.partial(
    pl.kernel,
    out_type=jax.ShapeDtypeStruct((NC, 2, NSEG, D), jnp.float32),
    mesh=plsc.VectorSubcoreMesh(core_axis_name="c", subcore_axis_name="s",
                                num_cores=NC, num_subcores=NS),
    scratch_types=[
        pltpu.VMEM((1, CHUNK, D), jnp.float32),
        pltpu.VMEM((1, CHUNK, D), jnp.float32),
        pltpu.VMEM((1, CHUNK), jnp.int32),
        pltpu.VMEM((SEG_PER_TILE, D), jnp.float32),
        pltpu.VMEM_SHARED((NSEG, D), jnp.float32),
        pltpu.VMEM_SHARED((NSEG, D), jnp.float32),
        pltpu.SemaphoreType.DMA,
        pltpu.SemaphoreType.DMA,
        pltpu.SemaphoreType.DMA,
    ],
)(_sc_body)


def _tc_body(part_ref, w_ref, b_ref, out_ref):
    p = part_ref[...]
    agg1 = p[0, 0] + p[1, 0]
    agg2 = p[0, 1] + p[1, 1]
    w = w_ref[...]
    out_ref[...] = (
        jnp.dot(agg1, w[:, :D].T, preferred_element_type=jnp.float32)
        + jnp.dot(agg2, w[:, D:].T, preferred_element_type=jnp.float32)
        + b_ref[...]
    )


def kernel(embs, prev_h, batch_idx, W, b):
    partials = _sc_segment_sums(prev_h, embs, batch_idx)
    out = pl.pallas_call(
        _tc_body,
        out_shape=jax.ShapeDtypeStruct((NSEG, D), jnp.float32),
    )(partials, W, b.reshape(1, D))
    return out


# speedup vs baseline: 6.6955x; 1.3550x over previous
import functools

import jax
import jax.numpy as jnp
from jax import lax
from jax.experimental import pallas as pl
from jax.experimental.pallas import tpu as pltpu
from jax.experimental.pallas import tpu_sc as plsc

NSEG = 1024
N = 320000
D = 128
NC = 2
NS = 16
NW = NC * NS
ROWS_PER_W = N // NW
CHUNK = 80
NCHUNK = ROWS_PER_W // CHUNK
SEG_PER_TILE = NSEG // NS


def _sc_body(prev_hbm, embs_hbm, idx_hbm, out_hbm,
             pbuf, ebuf, ibuf, zbuf, acc_p, acc_e,
             sem_p0, sem_e0, sem_i0, sem_p1, sem_e1, sem_i1):
    core = lax.axis_index("c")
    sid = lax.axis_index("s")
    wid = core * NS + sid
    row_base = wid * ROWS_PER_W
    sems = ((sem_p0, sem_e0, sem_i0), (sem_p1, sem_e1, sem_i1))

    zeros16 = jnp.zeros((16,), jnp.float32)

    def zero_row(r, _):
        for c in range(D // 16):
            zbuf[r, pl.ds(c * 16, 16)] = zeros16
        return _

    lax.fori_loop(0, SEG_PER_TILE, zero_row, None)
    seg_lo = sid * SEG_PER_TILE
    pltpu.sync_copy(zbuf, acc_p.at[pl.ds(seg_lo, SEG_PER_TILE)])
    pltpu.sync_copy(zbuf, acc_e.at[pl.ds(seg_lo, SEG_PER_TILE)])
    plsc.subcore_barrier()

    def issue(b, i):
        base = row_base + i * CHUNK
        sp, se, si = sems[b]
        pltpu.async_copy(prev_hbm.at[pl.ds(base, CHUNK)], pbuf.at[b], sp)
        pltpu.async_copy(embs_hbm.at[pl.ds(base, CHUNK)], ebuf.at[b], se)
        pltpu.async_copy(idx_hbm.at[pl.ds(base, CHUNK)], ibuf.at[b], si)

    def wait_dma(b):
        sp, se, si = sems[b]
        pltpu.make_async_copy(prev_hbm.at[pl.ds(0, CHUNK)], pbuf.at[b], sp).wait()
        pltpu.make_async_copy(embs_hbm.at[pl.ds(0, CHUNK)], ebuf.at[b], se).wait()
        pltpu.make_async_copy(idx_hbm.at[pl.ds(0, CHUNK)], ibuf.at[b], si).wait()

    issue(0, 0)
    issue(1, 1)

    def pair_body(g, _):
        def step(b):
            i = 2 * g + b

            @pl.when(i < NCHUNK)
            def _():
                wait_dma(b)
                pltpu.sync_copy(pbuf.at[b], acc_p.at[ibuf.at[b]], add=True)
                pltpu.sync_copy(ebuf.at[b], acc_e.at[ibuf.at[b]], add=True)

                @pl.when(i + 2 < NCHUNK)
                def _():
                    issue(b, i + 2)

        step(0)
        step(1)
        return _

    lax.fori_loop(0, (NCHUNK + 1) // 2, pair_body, None)

    plsc.subcore_barrier()
    pltpu.sync_copy(acc_p.at[pl.ds(seg_lo, SEG_PER_TILE)],
                    out_hbm.at[core, 0, pl.ds(seg_lo, SEG_PER_TILE)])
    pltpu.sync_copy(acc_e.at[pl.ds(seg_lo, SEG_PER_TILE)],
                    out_hbm.at[core, 1, pl.ds(seg_lo, SEG_PER_TILE)])


_sc_segment_sums = functools.partial(
    pl.kernel,
    out_type=jax.ShapeDtypeStruct((NC, 2, NSEG, D), jnp.float32),
    mesh=plsc.VectorSubcoreMesh(core_axis_name="c", subcore_axis_name="s",
                                num_cores=NC, num_subcores=NS),
    scratch_types=[
        pltpu.VMEM((2, CHUNK, D), jnp.float32),
        pltpu.VMEM((2, CHUNK, D), jnp.float32),
        pltpu.VMEM((2, CHUNK), jnp.int32),
        pltpu.VMEM((SEG_PER_TILE, D), jnp.float32),
        pltpu.VMEM_SHARED((NSEG, D), jnp.float32),
        pltpu.VMEM_SHARED((NSEG, D), jnp.float32),
        pltpu.SemaphoreType.DMA,
        pltpu.SemaphoreType.DMA,
        pltpu.SemaphoreType.DMA,
        pltpu.SemaphoreType.DMA,
        pltpu.SemaphoreType.DMA,
        pltpu.SemaphoreType.DMA,
    ],
)(_sc_body)


def _tc_body(part_ref, w_ref, b_ref, out_ref):
    p = part_ref[...]
    agg1 = p[0, 0] + p[1, 0]
    agg2 = p[0, 1] + p[1, 1]
    w = w_ref[...]
    out_ref[...] = (
        jnp.dot(agg1, w[:, :D].T, preferred_element_type=jnp.float32)
        + jnp.dot(agg2, w[:, D:].T, preferred_element_type=jnp.float32)
        + b_ref[...]
    )


def kernel(embs, prev_h, batch_idx, W, b):
    partials = _sc_segment_sums(prev_h, embs, batch_idx)
    out = pl.pallas_call(
        _tc_body,
        out_shape=jax.ShapeDtypeStruct((NSEG, D), jnp.float32),
    )(partials, W, b.reshape(1, D))
    return out
